# Initial kernel scaffold; baseline (speedup 1.0000x reference)
import jax
import jax.numpy as jnp
from jax.experimental import pallas as pl


def kernel(x, edge_index, W_pre, b_pre, W1, b1, W2, b2, W3, b3, W_head, b_head):
    raise NotImplementedError("write your pallas kernel here")


# speedup vs baseline: 10.9853x; 10.9853x over previous
import functools

import jax
import jax.numpy as jnp
from jax import lax
from jax.experimental import pallas as pl
from jax.experimental.pallas import tpu as pltpu
from jax.experimental.pallas import tpu_sc as plsc

_N = 10000
_E = 160000
_D = 256
_H = _D // 2
_NC = 2
_NS = 16
_ROWS_CH = 1000
_N_CH = _N // _ROWS_CH
_EPT_MSG = _E // _NS
_CH_MSG = 80
_NCHUNK = _EPT_MSG // _CH_MSG
_EPT_DEG = _E // (_NC * _NS)
_CH_DEG = 200
_BN = 1000


@functools.cache
def _sc_kernels():
    mesh = plsc.VectorSubcoreMesh(
        core_axis_name="c", subcore_axis_name="s",
        num_cores=_NC, num_subcores=_NS)

    @functools.partial(
        pl.kernel,
        out_type=jax.ShapeDtypeStruct((_NC, _N, _H), jnp.float32),
        mesh=mesh,
        scratch_types=[
            pltpu.VMEM((_CH_DEG,), jnp.int32),
            pltpu.VMEM((_CH_DEG, _H), jnp.float32),
            pltpu.VMEM_SHARED((_N, _H), jnp.float32),
        ],
    )
    def sc_degree(dst_hbm, ones_hbm, zeros_hbm, out_hbm, didx_v, ones_v, acc_sh):
        c = lax.axis_index("c")
        s = lax.axis_index("s")
        pltpu.sync_copy(ones_hbm, ones_v)

        @pl.when(s < _N_CH)
        def _zero():
            pltpu.sync_copy(zeros_hbm, acc_sh.at[pl.ds(s * _ROWS_CH, _ROWS_CH)])

        plsc.subcore_barrier()
        base = (c * _NS + s) * _EPT_DEG

        def body(k, carry):
            off = base + k * _CH_DEG
            pltpu.sync_copy(dst_hbm.at[pl.ds(off, _CH_DEG)], didx_v)
            pltpu.sync_copy(ones_v, acc_sh.at[didx_v], add=True)
            return carry

        lax.fori_loop(0, _EPT_DEG // _CH_DEG, body, 0)
        plsc.subcore_barrier()

        @pl.when(s < _N_CH)
        def _writeback():
            pltpu.sync_copy(acc_sh.at[pl.ds(s * _ROWS_CH, _ROWS_CH)],
                            out_hbm.at[c, pl.ds(s * _ROWS_CH, _ROWS_CH)])

    @functools.partial(
        pl.kernel,
        out_type=jax.ShapeDtypeStruct((_NC, _N, _H), jnp.float32),
        mesh=mesh,
        scratch_types=[
            pltpu.VMEM((_CH_MSG,), jnp.int32),
            pltpu.VMEM((_CH_MSG,), jnp.int32),
            pltpu.VMEM((_CH_MSG,), jnp.int32),
            pltpu.VMEM((_CH_MSG,), jnp.int32),
            pltpu.VMEM((_CH_MSG, _H), jnp.float32),
            pltpu.VMEM((_CH_MSG, _H), jnp.float32),
            pltpu.VMEM_SHARED((_N, _H), jnp.float32),
            pltpu.SemaphoreType.DMA,
            pltpu.SemaphoreType.DMA,
        ],
    )
    def sc_msgpass(g_hbm, srcadj_hbm, dst_hbm, zeros_hbm, out_hbm,
                   sidx0, sidx1, didx0, didx1, rows0, rows1, acc_sh,
                   sem0, sem1):
        c = lax.axis_index("c")
        s = lax.axis_index("s")

        @pl.when(s < _N_CH)
        def _zero():
            pltpu.sync_copy(zeros_hbm, acc_sh.at[pl.ds(s * _ROWS_CH, _ROWS_CH)])

        plsc.subcore_barrier()
        sbase = c * _E + s * _EPT_MSG
        dbase = s * _EPT_MSG

        def load_idx(chunk, sidx, didx):
            off = chunk * _CH_MSG
            pltpu.sync_copy(srcadj_hbm.at[pl.ds(sbase + off, _CH_MSG)], sidx)
            pltpu.sync_copy(dst_hbm.at[pl.ds(dbase + off, _CH_MSG)], didx)

        load_idx(0, sidx0, didx0)
        pltpu.async_copy(g_hbm.at[sidx0], rows0, sem0)

        def body(k2, carry):
            a = 2 * k2
            load_idx(a + 1, sidx1, didx1)
            pltpu.async_copy(g_hbm.at[sidx1], rows1, sem1)
            pltpu.make_async_copy(g_hbm.at[sidx0], rows0, sem0).wait()
            pltpu.sync_copy(rows0, acc_sh.at[didx0], add=True)
            load_idx(a + 2, sidx0, didx0)
            pltpu.async_copy(g_hbm.at[sidx0], rows0, sem0)
            pltpu.make_async_copy(g_hbm.at[sidx1], rows1, sem1).wait()
            pltpu.sync_copy(rows1, acc_sh.at[didx1], add=True)
            return carry

        lax.fori_loop(0, (_NCHUNK - 1) // 2, body, 0)
        pltpu.make_async_copy(g_hbm.at[sidx0], rows0, sem0).wait()
        pltpu.sync_copy(rows0, acc_sh.at[didx0], add=True)
        plsc.subcore_barrier()

        @pl.when(s < _N_CH)
        def _writeback():
            pltpu.sync_copy(acc_sh.at[pl.ds(s * _ROWS_CH, _ROWS_CH)],
                            out_hbm.at[c, pl.ds(s * _ROWS_CH, _ROWS_CH)])

    return sc_degree, sc_msgpass


def _pre_body(deg_ref, x_ref, Wp_ref, bp_ref, h_ref, g_ref, dinv_ref):
    dd = deg_ref[0][:, 0:1] + deg_ref[1][:, 0:1] + 1.0
    dv = lax.rsqrt(jnp.maximum(dd, 1.0))
    h = jnp.dot(x_ref[...], Wp_ref[...], preferred_element_type=jnp.float32)
    h = jnp.maximum(h + bp_ref[...], 0.0)
    h_ref[...] = h
    g = h * dv
    g_ref[0, :, :] = g[:, :_H]
    g_ref[1, :, :] = g[:, _H:]
    dinv_ref[...] = dv


def _tc_pre(deg_parts, x, W_pre, b_pre):
    return pl.pallas_call(
        _pre_body,
        grid=(_N // _BN,),
        in_specs=[
            pl.BlockSpec((_NC, _BN, _H), lambda i: (0, i, 0)),
            pl.BlockSpec((_BN, _D), lambda i: (i, 0)),
            pl.BlockSpec((_D, _D), lambda i: (0, 0)),
            pl.BlockSpec((1, _D), lambda i: (0, 0)),
        ],
        out_specs=[
            pl.BlockSpec((_BN, _D), lambda i: (i, 0)),
            pl.BlockSpec((_NC, _BN, _H), lambda i: (0, i, 0)),
            pl.BlockSpec((_BN, 1), lambda i: (i, 0)),
        ],
        out_shape=[
            jax.ShapeDtypeStruct((_N, _D), jnp.float32),
            jax.ShapeDtypeStruct((_NC, _N, _H), jnp.float32),
            jax.ShapeDtypeStruct((_N, 1), jnp.float32),
        ],
    )(deg_parts, x, W_pre, b_pre.reshape(1, _D))


def _layer_body(S_ref, h_ref, dinv_ref, W_ref, b_ref, h2_ref, g_ref):
    dv = dinv_ref[...]
    h = h_ref[...]
    dv2 = dv * dv
    z0 = S_ref[0] * dv + h[:, :_H] * dv2
    z1 = S_ref[1] * dv + h[:, _H:] * dv2
    acc = (jnp.dot(z0, W_ref[:_H, :], preferred_element_type=jnp.float32)
           + jnp.dot(z1, W_ref[_H:, :], preferred_element_type=jnp.float32))
    h2 = jnp.maximum(acc + b_ref[...], 0.0) + h
    h2_ref[...] = h2
    g = h2 * dv
    g_ref[0, :, :] = g[:, :_H]
    g_ref[1, :, :] = g[:, _H:]


def _tc_layer(S, h, dinv, W, b):
    return pl.pallas_call(
        _layer_body,
        grid=(_N // _BN,),
        in_specs=[
            pl.BlockSpec((_NC, _BN, _H), lambda i: (0, i, 0)),
            pl.BlockSpec((_BN, _D), lambda i: (i, 0)),
            pl.BlockSpec((_BN, 1), lambda i: (i, 0)),
            pl.BlockSpec((_D, _D), lambda i: (0, 0)),
            pl.BlockSpec((1, _D), lambda i: (0, 0)),
        ],
        out_specs=[
            pl.BlockSpec((_BN, _D), lambda i: (i, 0)),
            pl.BlockSpec((_NC, _BN, _H), lambda i: (0, i, 0)),
        ],
        out_shape=[
            jax.ShapeDtypeStruct((_N, _D), jnp.float32),
            jax.ShapeDtypeStruct((_NC, _N, _H), jnp.float32),
        ],
    )(S, h, dinv, W, b.reshape(1, _D))


def _tail_body(S_ref, h_ref, dinv_ref, W3_ref, b3_ref, Wh_ref, bh_ref, out_ref):
    dv = dinv_ref[...]
    h = h_ref[...]
    dv2 = dv * dv
    z0 = S_ref[0] * dv + h[:, :_H] * dv2
    z1 = S_ref[1] * dv + h[:, _H:] * dv2
    acc = (jnp.dot(z0, W3_ref[:_H, :], preferred_element_type=jnp.float32)
           + jnp.dot(z1, W3_ref[_H:, :], preferred_element_type=jnp.float32))
    h3 = jnp.maximum(acc + b3_ref[...], 0.0) + h
    out = jnp.dot(h3, Wh_ref[...], preferred_element_type=jnp.float32)
    out_ref[...] = out + bh_ref[...]


def _tc_tail(S, h, dinv, W3, b3, W_head, b_head):
    return pl.pallas_call(
        _tail_body,
        grid=(_N // _BN,),
        in_specs=[
            pl.BlockSpec((_NC, _BN, _H), lambda i: (0, i, 0)),
            pl.BlockSpec((_BN, _D), lambda i: (i, 0)),
            pl.BlockSpec((_BN, 1), lambda i: (i, 0)),
            pl.BlockSpec((_D, _D), lambda i: (0, 0)),
            pl.BlockSpec((1, _D), lambda i: (0, 0)),
            pl.BlockSpec((_D, _D), lambda i: (0, 0)),
            pl.BlockSpec((1, _D), lambda i: (0, 0)),
        ],
        out_specs=pl.BlockSpec((_BN, _D), lambda i: (i, 0)),
        out_shape=jax.ShapeDtypeStruct((_N, _D), jnp.float32),
    )(S, h, dinv, W3, b3.reshape(1, _D), W_head, b_head.reshape(1, _D))


def kernel(x, edge_index, W_pre, b_pre, W1, b1, W2, b2, W3, b3, W_head, b_head):
    src = edge_index[0]
    dst = edge_index[1]
    srcadj = jnp.concatenate([src, src + _N])
    onesH = jnp.ones((_CH_DEG, _H), jnp.float32)
    zerosH = jnp.zeros((_ROWS_CH, _H), jnp.float32)

    sc_degree, sc_msgpass = _sc_kernels()
    deg_parts = sc_degree(dst, onesH, zerosH)
    h, g, dinv = _tc_pre(deg_parts, x, W_pre, b_pre)
    for W, b in ((W1, b1), (W2, b2)):
        S = sc_msgpass(g.reshape(_NC * _N, _H), srcadj, dst, zerosH)
        h, g = _tc_layer(S, h, dinv, W, b)
    S = sc_msgpass(g.reshape(_NC * _N, _H), srcadj, dst, zerosH)
    return _tc_tail(S, h, dinv, W3, b3, W_head, b_head)


# speedup vs baseline: 15.0351x; 1.3687x over previous
import functools

import jax
import jax.numpy as jnp
from jax import lax
from jax.experimental import pallas as pl
from jax.experimental.pallas import tpu as pltpu
from jax.experimental.pallas import tpu_sc as plsc

_N = 10000
_E = 160000
_D = 256
_H = _D // 2
_NC = 2
_NS = 16
_ROWS_CH = 1000
_N_CH = _N // _ROWS_CH
_EPT_MSG = _E // _NS
_CH_MSG = 80
_NCHUNK = _EPT_MSG // _CH_MSG
_EPT_DEG = _E // (_NC * _NS)
_CH_DEG = 200
_BN = 1000


@functools.cache
def _sc_kernels():
    mesh = plsc.VectorSubcoreMesh(
        core_axis_name="c", subcore_axis_name="s",
        num_cores=_NC, num_subcores=_NS)

    @functools.partial(
        pl.kernel,
        out_type=jax.ShapeDtypeStruct((_NC, _N, _H), jnp.float32),
        mesh=mesh,
        scratch_types=[
            pltpu.VMEM((_CH_DEG,), jnp.int32),
            pltpu.VMEM((_CH_DEG, _H), jnp.float32),
            pltpu.VMEM_SHARED((_N, _H), jnp.float32),
        ],
    )
    def sc_degree(dst_hbm, ones_hbm, zeros_hbm, out_hbm, didx_v, ones_v, acc_sh):
        c = lax.axis_index("c")
        s = lax.axis_index("s")
        pltpu.sync_copy(ones_hbm, ones_v)

        @pl.when(s < _N_CH)
        def _zero():
            pltpu.sync_copy(zeros_hbm, acc_sh.at[pl.ds(s * _ROWS_CH, _ROWS_CH)])

        plsc.subcore_barrier()
        base = (c * _NS + s) * _EPT_DEG

        def body(k, carry):
            off = base + k * _CH_DEG
            pltpu.sync_copy(dst_hbm.at[pl.ds(off, _CH_DEG)], didx_v)
            pltpu.sync_copy(ones_v, acc_sh.at[didx_v], add=True)
            return carry

        lax.fori_loop(0, _EPT_DEG // _CH_DEG, body, 0)
        plsc.subcore_barrier()

        @pl.when(s < _N_CH)
        def _writeback():
            pltpu.sync_copy(acc_sh.at[pl.ds(s * _ROWS_CH, _ROWS_CH)],
                            out_hbm.at[c, pl.ds(s * _ROWS_CH, _ROWS_CH)])

    @functools.partial(
        pl.kernel,
        out_type=jax.ShapeDtypeStruct((_NC, _N, _H), jnp.float32),
        mesh=mesh,
        scratch_types=[
            pltpu.VMEM((_EPT_MSG,), jnp.int32),
            pltpu.VMEM((_EPT_MSG,), jnp.int32),
            pltpu.VMEM((_CH_MSG,), jnp.int32),
            pltpu.VMEM((_CH_MSG,), jnp.int32),
            pltpu.VMEM((_CH_MSG, _H), jnp.float32),
            pltpu.VMEM((_CH_MSG, _H), jnp.float32),
            pltpu.VMEM_SHARED((_N, _H), jnp.float32),
            pltpu.SemaphoreType.DMA,
            pltpu.SemaphoreType.DMA,
        ],
    )
    def sc_msgpass(g_hbm, srcadj_hbm, dst_hbm, zeros_hbm, out_hbm,
                   sidx_all, didx_all, didx0, didx1, rows0, rows1, acc_sh,
                   sem0, sem1):
        c = lax.axis_index("c")
        s = lax.axis_index("s")

        @pl.when(s < _N_CH)
        def _zero():
            pltpu.sync_copy(zeros_hbm, acc_sh.at[pl.ds(s * _ROWS_CH, _ROWS_CH)])

        pltpu.sync_copy(srcadj_hbm.at[pl.ds(c * _E + s * _EPT_MSG, _EPT_MSG)],
                        sidx_all)
        pltpu.sync_copy(dst_hbm.at[pl.ds(s * _EPT_MSG, _EPT_MSG)], didx_all)
        plsc.subcore_barrier()

        def start_gather(chunk, rows, sem):
            sl = sidx_all.at[pl.ds(chunk * _CH_MSG, _CH_MSG)]
            pltpu.async_copy(g_hbm.at[sl], rows, sem)

        def wait_gather(chunk, rows, sem):
            sl = sidx_all.at[pl.ds(chunk * _CH_MSG, _CH_MSG)]
            pltpu.make_async_copy(g_hbm.at[sl], rows, sem).wait()

        def stage_didx(chunk, didx):
            off = chunk * _CH_MSG
            for j in range(_CH_MSG // 16):
                didx[pl.ds(16 * j, 16)] = didx_all[pl.ds(off + 16 * j, 16)]

        stage_didx(0, didx0)
        start_gather(0, rows0, sem0)

        def body(k2, carry):
            a = 2 * k2
            stage_didx(a + 1, didx1)
            start_gather(a + 1, rows1, sem1)
            wait_gather(a, rows0, sem0)
            pltpu.sync_copy(rows0, acc_sh.at[didx0], add=True)
            stage_didx(a + 2, didx0)
            start_gather(a + 2, rows0, sem0)
            wait_gather(a + 1, rows1, sem1)
            pltpu.sync_copy(rows1, acc_sh.at[didx1], add=True)
            return carry

        lax.fori_loop(0, (_NCHUNK - 1) // 2, body, 0)
        wait_gather(_NCHUNK - 1, rows0, sem0)
        pltpu.sync_copy(rows0, acc_sh.at[didx0], add=True)
        plsc.subcore_barrier()

        @pl.when(s < _N_CH)
        def _writeback():
            pltpu.sync_copy(acc_sh.at[pl.ds(s * _ROWS_CH, _ROWS_CH)],
                            out_hbm.at[c, pl.ds(s * _ROWS_CH, _ROWS_CH)])

    return sc_degree, sc_msgpass


def _pre_body(deg_ref, x_ref, Wp_ref, bp_ref, h_ref, g_ref, dinv_ref):
    dd = deg_ref[0][:, 0:1] + deg_ref[1][:, 0:1] + 1.0
    dv = lax.rsqrt(jnp.maximum(dd, 1.0))
    h = jnp.dot(x_ref[...], Wp_ref[...], preferred_element_type=jnp.float32)
    h = jnp.maximum(h + bp_ref[...], 0.0)
    h_ref[...] = h
    g = h * dv
    g_ref[0, :, :] = g[:, :_H]
    g_ref[1, :, :] = g[:, _H:]
    dinv_ref[...] = dv


def _tc_pre(deg_parts, x, W_pre, b_pre):
    return pl.pallas_call(
        _pre_body,
        grid=(_N // _BN,),
        in_specs=[
            pl.BlockSpec((_NC, _BN, _H), lambda i: (0, i, 0)),
            pl.BlockSpec((_BN, _D), lambda i: (i, 0)),
            pl.BlockSpec((_D, _D), lambda i: (0, 0)),
            pl.BlockSpec((1, _D), lambda i: (0, 0)),
        ],
        out_specs=[
            pl.BlockSpec((_BN, _D), lambda i: (i, 0)),
            pl.BlockSpec((_NC, _BN, _H), lambda i: (0, i, 0)),
            pl.BlockSpec((_BN, 1), lambda i: (i, 0)),
        ],
        out_shape=[
            jax.ShapeDtypeStruct((_N, _D), jnp.float32),
            jax.ShapeDtypeStruct((_NC, _N, _H), jnp.float32),
            jax.ShapeDtypeStruct((_N, 1), jnp.float32),
        ],
    )(deg_parts, x, W_pre, b_pre.reshape(1, _D))


def _layer_body(S_ref, h_ref, dinv_ref, W_ref, b_ref, h2_ref, g_ref):
    dv = dinv_ref[...]
    h = h_ref[...]
    dv2 = dv * dv
    z0 = S_ref[0] * dv + h[:, :_H] * dv2
    z1 = S_ref[1] * dv + h[:, _H:] * dv2
    acc = (jnp.dot(z0, W_ref[:_H, :], preferred_element_type=jnp.float32)
           + jnp.dot(z1, W_ref[_H:, :], preferred_element_type=jnp.float32))
    h2 = jnp.maximum(acc + b_ref[...], 0.0) + h
    h2_ref[...] = h2
    g = h2 * dv
    g_ref[0, :, :] = g[:, :_H]
    g_ref[1, :, :] = g[:, _H:]


def _tc_layer(S, h, dinv, W, b):
    return pl.pallas_call(
        _layer_body,
        grid=(_N // _BN,),
        in_specs=[
            pl.BlockSpec((_NC, _BN, _H), lambda i: (0, i, 0)),
            pl.BlockSpec((_BN, _D), lambda i: (i, 0)),
            pl.BlockSpec((_BN, 1), lambda i: (i, 0)),
            pl.BlockSpec((_D, _D), lambda i: (0, 0)),
            pl.BlockSpec((1, _D), lambda i: (0, 0)),
        ],
        out_specs=[
            pl.BlockSpec((_BN, _D), lambda i: (i, 0)),
            pl.BlockSpec((_NC, _BN, _H), lambda i: (0, i, 0)),
        ],
        out_shape=[
            jax.ShapeDtypeStruct((_N, _D), jnp.float32),
            jax.ShapeDtypeStruct((_NC, _N, _H), jnp.float32),
        ],
    )(S, h, dinv, W, b.reshape(1, _D))


def _tail_body(S_ref, h_ref, dinv_ref, W3_ref, b3_ref, Wh_ref, bh_ref, out_ref):
    dv = dinv_ref[...]
    h = h_ref[...]
    dv2 = dv * dv
    z0 = S_ref[0] * dv + h[:, :_H] * dv2
    z1 = S_ref[1] * dv + h[:, _H:] * dv2
    acc = (jnp.dot(z0, W3_ref[:_H, :], preferred_element_type=jnp.float32)
           + jnp.dot(z1, W3_ref[_H:, :], preferred_element_type=jnp.float32))
    h3 = jnp.maximum(acc + b3_ref[...], 0.0) + h
    out = jnp.dot(h3, Wh_ref[...], preferred_element_type=jnp.float32)
    out_ref[...] = out + bh_ref[...]


def _tc_tail(S, h, dinv, W3, b3, W_head, b_head):
    return pl.pallas_call(
        _tail_body,
        grid=(_N // _BN,),
        in_specs=[
            pl.BlockSpec((_NC, _BN, _H), lambda i: (0, i, 0)),
            pl.BlockSpec((_BN, _D), lambda i: (i, 0)),
            pl.BlockSpec((_BN, 1), lambda i: (i, 0)),
            pl.BlockSpec((_D, _D), lambda i: (0, 0)),
            pl.BlockSpec((1, _D), lambda i: (0, 0)),
            pl.BlockSpec((_D, _D), lambda i: (0, 0)),
            pl.BlockSpec((1, _D), lambda i: (0, 0)),
        ],
        out_specs=pl.BlockSpec((_BN, _D), lambda i: (i, 0)),
        out_shape=jax.ShapeDtypeStruct((_N, _D), jnp.float32),
    )(S, h, dinv, W3, b3.reshape(1, _D), W_head, b_head.reshape(1, _D))


def kernel(x, edge_index, W_pre, b_pre, W1, b1, W2, b2, W3, b3, W_head, b_head):
    src = edge_index[0]
    dst = edge_index[1]
    srcadj = jnp.concatenate([src, src + _N])
    onesH = jnp.ones((_CH_DEG, _H), jnp.float32)
    zerosH = jnp.zeros((_ROWS_CH, _H), jnp.float32)

    sc_degree, sc_msgpass = _sc_kernels()
    deg_parts = sc_degree(dst, onesH, zerosH)
    h, g, dinv = _tc_pre(deg_parts, x, W_pre, b_pre)
    for W, b in ((W1, b1), (W2, b2)):
        S = sc_msgpass(g.reshape(_NC * _N, _H), srcadj, dst, zerosH)
        h, g = _tc_layer(S, h, dinv, W, b)
    S = sc_msgpass(g.reshape(_NC * _N, _H), srcadj, dst, zerosH)
    return _tc_tail(S, h, dinv, W3, b3, W_head, b_head)


# speedup vs baseline: 16.5662x; 1.1018x over previous
import functools

import jax
import jax.numpy as jnp
from jax import lax
from jax.experimental import pallas as pl
from jax.experimental.pallas import tpu as pltpu
from jax.experimental.pallas import tpu_sc as plsc

_N = 10000
_E = 160000
_D = 256
_H = _D // 2
_NC = 2
_NS = 16
_ROWS_CH = 1000
_N_CH = _N // _ROWS_CH
_EPT_MSG = _E // _NS
_CH_MSG = 80
_NCHUNK = _EPT_MSG // _CH_MSG
_EPT_DEG = _E // (_NC * _NS)
_NRED = 5
_N_PAD = 10240
_BN = 1000


@functools.cache
def _sc_kernels():
    mesh = plsc.VectorSubcoreMesh(
        core_axis_name="c", subcore_axis_name="s",
        num_cores=_NC, num_subcores=_NS)

    @functools.partial(
        pl.kernel,
        out_type=jax.ShapeDtypeStruct((_NC, _N_PAD), jnp.float32),
        mesh=mesh,
        compiler_params=pltpu.CompilerParams(needs_layout_passes=False),
        scratch_types=[
            pltpu.VMEM((_EPT_DEG + 16,), jnp.int32),
            pltpu.VMEM((_N_PAD,), jnp.float32),
            pltpu.VMEM((_NS, _N_PAD // _NRED), jnp.float32),
            pltpu.VMEM((_N_PAD // _NRED,), jnp.float32),
            pltpu.VMEM_SHARED((_NS, _N_PAD), jnp.float32),
        ],
    )
    def sc_degree(dst_hbm, out_hbm, didx_all, hist, rbuf, red, stage_sh):
        c = lax.axis_index("c")
        s = lax.axis_index("s")
        pltpu.sync_copy(dst_hbm.at[pl.ds((c * _NS + s) * _EPT_DEG, _EPT_DEG)],
                        didx_all.at[pl.ds(0, _EPT_DEG)])
        zeros16 = jnp.zeros((16,), jnp.float32)
        ones16 = jnp.ones((16,), jnp.float32)

        def zbody(i, carry):
            hist[pl.ds(16 * i, 16)] = zeros16
            return carry

        lax.fori_loop(0, _N_PAD // 16, zbody, 0)

        def sbody(k, carry):
            idx = didx_all[pl.ds(16 * k, 16)]
            plsc.addupdate_scatter(hist, [idx], ones16)
            return carry

        nfull = _EPT_DEG // 16
        lax.fori_loop(0, nfull, sbody, 0)
        rem = _EPT_DEG - nfull * 16
        if rem:
            idx = didx_all[pl.ds(16 * nfull, 16)]
            mask = lax.iota(jnp.int32, 16) < rem
            plsc.addupdate_scatter(hist, [jnp.where(mask, idx, 0)], ones16,
                                   mask=mask)
        pltpu.sync_copy(hist, stage_sh.at[s])
        plsc.subcore_barrier()

        @pl.when(s < _NRED)
        def _reduce():
            cols = _N_PAD // _NRED
            pltpu.sync_copy(stage_sh.at[:, pl.ds(s * cols, cols)], rbuf)

            def rbody(j, carry):
                acc = rbuf[0, pl.ds(16 * j, 16)]
                for r in range(1, _NS):
                    acc = acc + rbuf[r, pl.ds(16 * j, 16)]
                red[pl.ds(16 * j, 16)] = acc
                return carry

            lax.fori_loop(0, cols // 16, rbody, 0)
            pltpu.sync_copy(red, out_hbm.at[c, pl.ds(s * cols, cols)])

    @functools.partial(
        pl.kernel,
        out_type=jax.ShapeDtypeStruct((_NC, _N, _H), jnp.float32),
        mesh=mesh,
        scratch_types=[
            pltpu.VMEM((_EPT_MSG,), jnp.int32),
            pltpu.VMEM((_EPT_MSG,), jnp.int32),
            pltpu.VMEM((_CH_MSG,), jnp.int32),
            pltpu.VMEM((_CH_MSG,), jnp.int32),
            pltpu.VMEM((_CH_MSG, _H), jnp.float32),
            pltpu.VMEM((_CH_MSG, _H), jnp.float32),
            pltpu.VMEM_SHARED((_N, _H), jnp.float32),
            pltpu.SemaphoreType.DMA,
            pltpu.SemaphoreType.DMA,
        ],
    )
    def sc_msgpass(g_hbm, srcadj_hbm, dst_hbm, zeros_hbm, out_hbm,
                   sidx_all, didx_all, didx0, didx1, rows0, rows1, acc_sh,
                   sem0, sem1):
        c = lax.axis_index("c")
        s = lax.axis_index("s")

        @pl.when(s < _N_CH)
        def _zero():
            pltpu.sync_copy(zeros_hbm, acc_sh.at[pl.ds(s * _ROWS_CH, _ROWS_CH)])

        pltpu.sync_copy(srcadj_hbm.at[pl.ds(c * _E + s * _EPT_MSG, _EPT_MSG)],
                        sidx_all)
        pltpu.sync_copy(dst_hbm.at[pl.ds(s * _EPT_MSG, _EPT_MSG)], didx_all)
        plsc.subcore_barrier()

        def start_gather(chunk, rows, sem):
            sl = sidx_all.at[pl.ds(chunk * _CH_MSG, _CH_MSG)]
            pltpu.async_copy(g_hbm.at[sl], rows, sem)

        def wait_gather(chunk, rows, sem):
            sl = sidx_all.at[pl.ds(chunk * _CH_MSG, _CH_MSG)]
            pltpu.make_async_copy(g_hbm.at[sl], rows, sem).wait()

        def stage_didx(chunk, didx):
            off = chunk * _CH_MSG
            for j in range(_CH_MSG // 16):
                didx[pl.ds(16 * j, 16)] = didx_all[pl.ds(off + 16 * j, 16)]

        stage_didx(0, didx0)
        start_gather(0, rows0, sem0)

        def body(k2, carry):
            a = 2 * k2
            stage_didx(a + 1, didx1)
            start_gather(a + 1, rows1, sem1)
            wait_gather(a, rows0, sem0)
            pltpu.sync_copy(rows0, acc_sh.at[didx0], add=True)
            stage_didx(a + 2, didx0)
            start_gather(a + 2, rows0, sem0)
            wait_gather(a + 1, rows1, sem1)
            pltpu.sync_copy(rows1, acc_sh.at[didx1], add=True)
            return carry

        lax.fori_loop(0, (_NCHUNK - 1) // 2, body, 0)
        wait_gather(_NCHUNK - 1, rows0, sem0)
        pltpu.sync_copy(rows0, acc_sh.at[didx0], add=True)
        plsc.subcore_barrier()

        @pl.when(s < _N_CH)
        def _writeback():
            pltpu.sync_copy(acc_sh.at[pl.ds(s * _ROWS_CH, _ROWS_CH)],
                            out_hbm.at[c, pl.ds(s * _ROWS_CH, _ROWS_CH)])

    return sc_degree, sc_msgpass


def _pre_body(deg_ref, x_ref, Wp_ref, bp_ref, h_ref, g_ref, dinv_ref):
    dd = deg_ref[...] + 1.0
    dv = lax.rsqrt(jnp.maximum(dd, 1.0))
    h = jnp.dot(x_ref[...], Wp_ref[...], preferred_element_type=jnp.float32)
    h = jnp.maximum(h + bp_ref[...], 0.0)
    h_ref[...] = h
    g = h * dv
    g_ref[0, :, :] = g[:, :_H]
    g_ref[1, :, :] = g[:, _H:]
    dinv_ref[...] = dv


def _tc_pre(deg_parts, x, W_pre, b_pre):
    return pl.pallas_call(
        _pre_body,
        grid=(_N // _BN,),
        in_specs=[
            pl.BlockSpec((_BN, 1), lambda i: (i, 0)),
            pl.BlockSpec((_BN, _D), lambda i: (i, 0)),
            pl.BlockSpec((_D, _D), lambda i: (0, 0)),
            pl.BlockSpec((1, _D), lambda i: (0, 0)),
        ],
        out_specs=[
            pl.BlockSpec((_BN, _D), lambda i: (i, 0)),
            pl.BlockSpec((_NC, _BN, _H), lambda i: (0, i, 0)),
            pl.BlockSpec((_BN, 1), lambda i: (i, 0)),
        ],
        out_shape=[
            jax.ShapeDtypeStruct((_N, _D), jnp.float32),
            jax.ShapeDtypeStruct((_NC, _N, _H), jnp.float32),
            jax.ShapeDtypeStruct((_N, 1), jnp.float32),
        ],
    )(deg_parts, x, W_pre, b_pre.reshape(1, _D))


def _layer_body(S_ref, h_ref, dinv_ref, W_ref, b_ref, h2_ref, g_ref):
    dv = dinv_ref[...]
    h = h_ref[...]
    dv2 = dv * dv
    z0 = S_ref[0] * dv + h[:, :_H] * dv2
    z1 = S_ref[1] * dv + h[:, _H:] * dv2
    acc = (jnp.dot(z0, W_ref[:_H, :], preferred_element_type=jnp.float32)
           + jnp.dot(z1, W_ref[_H:, :], preferred_element_type=jnp.float32))
    h2 = jnp.maximum(acc + b_ref[...], 0.0) + h
    h2_ref[...] = h2
    g = h2 * dv
    g_ref[0, :, :] = g[:, :_H]
    g_ref[1, :, :] = g[:, _H:]


def _tc_layer(S, h, dinv, W, b):
    return pl.pallas_call(
        _layer_body,
        grid=(_N // _BN,),
        in_specs=[
            pl.BlockSpec((_NC, _BN, _H), lambda i: (0, i, 0)),
            pl.BlockSpec((_BN, _D), lambda i: (i, 0)),
            pl.BlockSpec((_BN, 1), lambda i: (i, 0)),
            pl.BlockSpec((_D, _D), lambda i: (0, 0)),
            pl.BlockSpec((1, _D), lambda i: (0, 0)),
        ],
        out_specs=[
            pl.BlockSpec((_BN, _D), lambda i: (i, 0)),
            pl.BlockSpec((_NC, _BN, _H), lambda i: (0, i, 0)),
        ],
        out_shape=[
            jax.ShapeDtypeStruct((_N, _D), jnp.float32),
            jax.ShapeDtypeStruct((_NC, _N, _H), jnp.float32),
        ],
    )(S, h, dinv, W, b.reshape(1, _D))


def _tail_body(S_ref, h_ref, dinv_ref, W3_ref, b3_ref, Wh_ref, bh_ref, out_ref):
    dv = dinv_ref[...]
    h = h_ref[...]
    dv2 = dv * dv
    z0 = S_ref[0] * dv + h[:, :_H] * dv2
    z1 = S_ref[1] * dv + h[:, _H:] * dv2
    acc = (jnp.dot(z0, W3_ref[:_H, :], preferred_element_type=jnp.float32)
           + jnp.dot(z1, W3_ref[_H:, :], preferred_element_type=jnp.float32))
    h3 = jnp.maximum(acc + b3_ref[...], 0.0) + h
    out = jnp.dot(h3, Wh_ref[...], preferred_element_type=jnp.float32)
    out_ref[...] = out + bh_ref[...]


def _tc_tail(S, h, dinv, W3, b3, W_head, b_head):
    return pl.pallas_call(
        _tail_body,
        grid=(_N // _BN,),
        in_specs=[
            pl.BlockSpec((_NC, _BN, _H), lambda i: (0, i, 0)),
            pl.BlockSpec((_BN, _D), lambda i: (i, 0)),
            pl.BlockSpec((_BN, 1), lambda i: (i, 0)),
            pl.BlockSpec((_D, _D), lambda i: (0, 0)),
            pl.BlockSpec((1, _D), lambda i: (0, 0)),
            pl.BlockSpec((_D, _D), lambda i: (0, 0)),
            pl.BlockSpec((1, _D), lambda i: (0, 0)),
        ],
        out_specs=pl.BlockSpec((_BN, _D), lambda i: (i, 0)),
        out_shape=jax.ShapeDtypeStruct((_N, _D), jnp.float32),
    )(S, h, dinv, W3, b3.reshape(1, _D), W_head, b_head.reshape(1, _D))


def kernel(x, edge_index, W_pre, b_pre, W1, b1, W2, b2, W3, b3, W_head, b_head):
    src = edge_index[0]
    dst = edge_index[1]
    srcadj = jnp.concatenate([src, src + _N])
    zerosH = jnp.zeros((_ROWS_CH, _H), jnp.float32)

    sc_degree, sc_msgpass = _sc_kernels()
    deg_parts = sc_degree(dst)
    deg_col = (deg_parts[0, :_N] + deg_parts[1, :_N]).reshape(_N, 1)
    h, g, dinv = _tc_pre(deg_col, x, W_pre, b_pre)
    for W, b in ((W1, b1), (W2, b2)):
        S = sc_msgpass(g.reshape(_NC * _N, _H), srcadj, dst, zerosH)
        h, g = _tc_layer(S, h, dinv, W, b)
    S = sc_msgpass(g.reshape(_NC * _N, _H), srcadj, dst, zerosH)
    return _tc_tail(S, h, dinv, W3, b3, W_head, b_head)


# speedup vs baseline: 16.5933x; 1.0016x over previous
import functools

import jax
import jax.numpy as jnp
from jax import lax
from jax.experimental import pallas as pl
from jax.experimental.pallas import tpu as pltpu
from jax.experimental.pallas import tpu_sc as plsc

_N = 10000
_E = 160000
_D = 256
_H = _D // 2
_NC = 2
_NS = 16
_ROWS_CH = 1000
_N_CH = _N // _ROWS_CH
_EPT_MSG = _E // _NS
_CH_MSG = 80
_NCHUNK = _EPT_MSG // _CH_MSG
_EPT_DEG = _E // (_NC * _NS)
_NRED = 5
_N_PAD = 10240
_BN = 1000


@functools.cache
def _sc_kernels():
    mesh = plsc.VectorSubcoreMesh(
        core_axis_name="c", subcore_axis_name="s",
        num_cores=_NC, num_subcores=_NS)

    @functools.partial(
        pl.kernel,
        out_type=jax.ShapeDtypeStruct((_NC, _N_PAD), jnp.float32),
        mesh=mesh,
        compiler_params=pltpu.CompilerParams(needs_layout_passes=False),
        scratch_types=[
            pltpu.VMEM((_EPT_DEG + 16,), jnp.int32),
            pltpu.VMEM((_N_PAD,), jnp.float32),
            pltpu.VMEM((_NS, _N_PAD // _NRED), jnp.float32),
            pltpu.VMEM((_N_PAD // _NRED,), jnp.float32),
            pltpu.VMEM_SHARED((_NS, _N_PAD), jnp.float32),
        ],
    )
    def sc_degree(dst_hbm, out_hbm, didx_all, hist, rbuf, red, stage_sh):
        c = lax.axis_index("c")
        s = lax.axis_index("s")
        pltpu.sync_copy(dst_hbm.at[pl.ds((c * _NS + s) * _EPT_DEG, _EPT_DEG)],
                        didx_all.at[pl.ds(0, _EPT_DEG)])
        zeros16 = jnp.zeros((16,), jnp.float32)
        ones16 = jnp.ones((16,), jnp.float32)

        def zbody(i, carry):
            hist[pl.ds(16 * i, 16)] = zeros16
            return carry

        lax.fori_loop(0, _N_PAD // 16, zbody, 0)

        def sbody(k, carry):
            idx = didx_all[pl.ds(16 * k, 16)]
            plsc.addupdate_scatter(hist, [idx], ones16)
            return carry

        nfull = _EPT_DEG // 16
        lax.fori_loop(0, nfull, sbody, 0)
        rem = _EPT_DEG - nfull * 16
        if rem:
            idx = didx_all[pl.ds(16 * nfull, 16)]
            mask = lax.iota(jnp.int32, 16) < rem
            plsc.addupdate_scatter(hist, [jnp.where(mask, idx, 0)], ones16,
                                   mask=mask)
        pltpu.sync_copy(hist, stage_sh.at[s])
        plsc.subcore_barrier()

        @pl.when(s < _NRED)
        def _reduce():
            cols = _N_PAD // _NRED
            pltpu.sync_copy(stage_sh.at[:, pl.ds(s * cols, cols)], rbuf)

            def rbody(j, carry):
                acc = rbuf[0, pl.ds(16 * j, 16)]
                for r in range(1, _NS):
                    acc = acc + rbuf[r, pl.ds(16 * j, 16)]
                red[pl.ds(16 * j, 16)] = acc
                return carry

            lax.fori_loop(0, cols // 16, rbody, 0)
            pltpu.sync_copy(red, out_hbm.at[c, pl.ds(s * cols, cols)])

    @functools.partial(
        pl.kernel,
        out_type=jax.ShapeDtypeStruct((_NC, _N, _H), jnp.float32),
        mesh=mesh,
        scratch_types=[
            pltpu.VMEM((_EPT_MSG,), jnp.int32),
            pltpu.VMEM((_EPT_MSG,), jnp.int32),
            pltpu.VMEM((_CH_MSG,), jnp.int32),
            pltpu.VMEM((_CH_MSG,), jnp.int32),
            pltpu.VMEM((_CH_MSG, _H), jnp.float32),
            pltpu.VMEM((_CH_MSG, _H), jnp.float32),
            pltpu.VMEM_SHARED((_N, _H), jnp.float32),
            pltpu.SemaphoreType.DMA,
            pltpu.SemaphoreType.DMA,
        ],
    )
    def sc_msgpass(g_hbm, srcadj_hbm, dst_hbm, zeros_hbm, out_hbm,
                   sidx_all, didx_all, didx0, didx1, rows0, rows1, acc_sh,
                   sem0, sem1):
        c = lax.axis_index("c")
        s = lax.axis_index("s")

        @pl.when(s < _N_CH)
        def _zero():
            pltpu.sync_copy(zeros_hbm, acc_sh.at[pl.ds(s * _ROWS_CH, _ROWS_CH)])

        pltpu.sync_copy(srcadj_hbm.at[pl.ds(c * _E + s * _EPT_MSG, _EPT_MSG)],
                        sidx_all)
        pltpu.sync_copy(dst_hbm.at[pl.ds(s * _EPT_MSG, _EPT_MSG)], didx_all)
        plsc.subcore_barrier()

        def start_gather(chunk, rows, sem):
            sl = sidx_all.at[pl.ds(chunk * _CH_MSG, _CH_MSG)]
            pltpu.async_copy(g_hbm.at[sl], rows, sem)

        def wait_gather(chunk, rows, sem):
            sl = sidx_all.at[pl.ds(chunk * _CH_MSG, _CH_MSG)]
            pltpu.make_async_copy(g_hbm.at[sl], rows, sem).wait()

        def stage_didx(chunk, didx):
            off = chunk * _CH_MSG
            for j in range(_CH_MSG // 16):
                didx[pl.ds(16 * j, 16)] = didx_all[pl.ds(off + 16 * j, 16)]

        stage_didx(0, didx0)
        start_gather(0, rows0, sem0)

        def body(k2, carry):
            a = 2 * k2
            stage_didx(a + 1, didx1)
            start_gather(a + 1, rows1, sem1)
            wait_gather(a, rows0, sem0)
            pltpu.sync_copy(rows0, acc_sh.at[didx0], add=True)
            stage_didx(a + 2, didx0)
            start_gather(a + 2, rows0, sem0)
            wait_gather(a + 1, rows1, sem1)
            pltpu.sync_copy(rows1, acc_sh.at[didx1], add=True)
            return carry

        lax.fori_loop(0, (_NCHUNK - 1) // 2, body, 0)
        wait_gather(_NCHUNK - 1, rows0, sem0)
        pltpu.sync_copy(rows0, acc_sh.at[didx0], add=True)
        plsc.subcore_barrier()

        @pl.when(s < _N_CH)
        def _writeback():
            pltpu.sync_copy(acc_sh.at[pl.ds(s * _ROWS_CH, _ROWS_CH)],
                            out_hbm.at[c, pl.ds(s * _ROWS_CH, _ROWS_CH)])

    return sc_degree, sc_msgpass


def _mm_body(x_ref, Wp_ref, bp_ref, h_ref):
    h = jnp.dot(x_ref[...], Wp_ref[...], preferred_element_type=jnp.float32)
    h_ref[...] = jnp.maximum(h + bp_ref[...], 0.0)


def _tc_premm(x, W_pre, b_pre):
    return pl.pallas_call(
        _mm_body,
        grid=(_N // _BN,),
        in_specs=[
            pl.BlockSpec((_BN, _D), lambda i: (i, 0)),
            pl.BlockSpec((_D, _D), lambda i: (0, 0)),
            pl.BlockSpec((1, _D), lambda i: (0, 0)),
        ],
        out_specs=pl.BlockSpec((_BN, _D), lambda i: (i, 0)),
        out_shape=jax.ShapeDtypeStruct((_N, _D), jnp.float32),
    )(x, W_pre, b_pre.reshape(1, _D))


def _g_body(deg_ref, h_ref, g_ref, dinv_ref):
    dd = deg_ref[...] + 1.0
    dv = lax.rsqrt(jnp.maximum(dd, 1.0))
    h = h_ref[...]
    g = h * dv
    g_ref[0, :, :] = g[:, :_H]
    g_ref[1, :, :] = g[:, _H:]
    dinv_ref[...] = dv


def _tc_g(deg_col, h):
    return pl.pallas_call(
        _g_body,
        grid=(_N // _BN,),
        in_specs=[
            pl.BlockSpec((_BN, 1), lambda i: (i, 0)),
            pl.BlockSpec((_BN, _D), lambda i: (i, 0)),
        ],
        out_specs=[
            pl.BlockSpec((_NC, _BN, _H), lambda i: (0, i, 0)),
            pl.BlockSpec((_BN, 1), lambda i: (i, 0)),
        ],
        out_shape=[
            jax.ShapeDtypeStruct((_NC, _N, _H), jnp.float32),
            jax.ShapeDtypeStruct((_N, 1), jnp.float32),
        ],
    )(deg_col, h)


def _layer_body(S_ref, h_ref, dinv_ref, W_ref, b_ref, h2_ref, g_ref):
    dv = dinv_ref[...]
    h = h_ref[...]
    dv2 = dv * dv
    z0 = S_ref[0] * dv + h[:, :_H] * dv2
    z1 = S_ref[1] * dv + h[:, _H:] * dv2
    acc = (jnp.dot(z0, W_ref[:_H, :], preferred_element_type=jnp.float32)
           + jnp.dot(z1, W_ref[_H:, :], preferred_element_type=jnp.float32))
    h2 = jnp.maximum(acc + b_ref[...], 0.0) + h
    h2_ref[...] = h2
    g = h2 * dv
    g_ref[0, :, :] = g[:, :_H]
    g_ref[1, :, :] = g[:, _H:]


def _tc_layer(S, h, dinv, W, b):
    return pl.pallas_call(
        _layer_body,
        grid=(_N // _BN,),
        in_specs=[
            pl.BlockSpec((_NC, _BN, _H), lambda i: (0, i, 0)),
            pl.BlockSpec((_BN, _D), lambda i: (i, 0)),
            pl.BlockSpec((_BN, 1), lambda i: (i, 0)),
            pl.BlockSpec((_D, _D), lambda i: (0, 0)),
            pl.BlockSpec((1, _D), lambda i: (0, 0)),
        ],
        out_specs=[
            pl.BlockSpec((_BN, _D), lambda i: (i, 0)),
            pl.BlockSpec((_NC, _BN, _H), lambda i: (0, i, 0)),
        ],
        out_shape=[
            jax.ShapeDtypeStruct((_N, _D), jnp.float32),
            jax.ShapeDtypeStruct((_NC, _N, _H), jnp.float32),
        ],
    )(S, h, dinv, W, b.reshape(1, _D))


def _tail_body(S_ref, h_ref, dinv_ref, W3_ref, b3_ref, Wh_ref, bh_ref, out_ref):
    dv = dinv_ref[...]
    h = h_ref[...]
    dv2 = dv * dv
    z0 = S_ref[0] * dv + h[:, :_H] * dv2
    z1 = S_ref[1] * dv + h[:, _H:] * dv2
    acc = (jnp.dot(z0, W3_ref[:_H, :], preferred_element_type=jnp.float32)
           + jnp.dot(z1, W3_ref[_H:, :], preferred_element_type=jnp.float32))
    h3 = jnp.maximum(acc + b3_ref[...], 0.0) + h
    out = jnp.dot(h3, Wh_ref[...], preferred_element_type=jnp.float32)
    out_ref[...] = out + bh_ref[...]


def _tc_tail(S, h, dinv, W3, b3, W_head, b_head):
    return pl.pallas_call(
        _tail_body,
        grid=(_N // _BN,),
        in_specs=[
            pl.BlockSpec((_NC, _BN, _H), lambda i: (0, i, 0)),
            pl.BlockSpec((_BN, _D), lambda i: (i, 0)),
            pl.BlockSpec((_BN, 1), lambda i: (i, 0)),
            pl.BlockSpec((_D, _D), lambda i: (0, 0)),
            pl.BlockSpec((1, _D), lambda i: (0, 0)),
            pl.BlockSpec((_D, _D), lambda i: (0, 0)),
            pl.BlockSpec((1, _D), lambda i: (0, 0)),
        ],
        out_specs=pl.BlockSpec((_BN, _D), lambda i: (i, 0)),
        out_shape=jax.ShapeDtypeStruct((_N, _D), jnp.float32),
    )(S, h, dinv, W3, b3.reshape(1, _D), W_head, b_head.reshape(1, _D))


def kernel(x, edge_index, W_pre, b_pre, W1, b1, W2, b2, W3, b3, W_head, b_head):
    src = edge_index[0]
    dst = edge_index[1]
    srcadj = jnp.concatenate([src, src + _N])
    zerosH = jnp.zeros((_ROWS_CH, _H), jnp.float32)

    sc_degree, sc_msgpass = _sc_kernels()
    deg_parts = sc_degree(dst)
    h = _tc_premm(x, W_pre, b_pre)
    deg_col = (deg_parts[0, :_N] + deg_parts[1, :_N]).reshape(_N, 1)
    g, dinv = _tc_g(deg_col, h)
    for W, b in ((W1, b1), (W2, b2)):
        S = sc_msgpass(g.reshape(_NC * _N, _H), srcadj, dst, zerosH)
        h, g = _tc_layer(S, h, dinv, W, b)
    S = sc_msgpass(g.reshape(_NC * _N, _H), srcadj, dst, zerosH)
    return _tc_tail(S, h, dinv, W3, b3, W_head, b_head)


# speedup vs baseline: 16.9309x; 1.0203x over previous
import functools

import jax
import jax.numpy as jnp
from jax import lax
from jax.experimental import pallas as pl
from jax.experimental.pallas import tpu as pltpu
from jax.experimental.pallas import tpu_sc as plsc

_N = 10000
_E = 160000
_D = 256
_H = _D // 2
_NC = 2
_NS = 16
_ROWS_CH = 1000
_N_CH = _N // _ROWS_CH
_EPT_MSG = _E // _NS
_CH_MSG = 80
_NCHUNK = _EPT_MSG // _CH_MSG
_EPT_DEG = _E // (_NC * _NS)
_NRED = 5
_N_PAD = 10240
_BN = 1000


@functools.cache
def _sc_kernels():
    mesh = plsc.VectorSubcoreMesh(
        core_axis_name="c", subcore_axis_name="s",
        num_cores=_NC, num_subcores=_NS)

    @functools.partial(
        pl.kernel,
        out_type=jax.ShapeDtypeStruct((_NC, _N_PAD), jnp.float32),
        mesh=mesh,
        compiler_params=pltpu.CompilerParams(needs_layout_passes=False),
        scratch_types=[
            pltpu.VMEM((_EPT_DEG + 16,), jnp.int32),
            pltpu.VMEM((_N_PAD,), jnp.float32),
            pltpu.VMEM((_NS, _N_PAD // _NRED), jnp.float32),
            pltpu.VMEM((_N_PAD // _NRED,), jnp.float32),
            pltpu.VMEM_SHARED((_NS, _N_PAD), jnp.float32),
        ],
    )
    def sc_degree(dst_hbm, out_hbm, didx_all, hist, rbuf, red, stage_sh):
        c = lax.axis_index("c")
        s = lax.axis_index("s")
        pltpu.sync_copy(dst_hbm.at[pl.ds((c * _NS + s) * _EPT_DEG, _EPT_DEG)],
                        didx_all.at[pl.ds(0, _EPT_DEG)])
        zeros16 = jnp.zeros((16,), jnp.float32)
        ones16 = jnp.ones((16,), jnp.float32)

        def zbody(i, carry):
            hist[pl.ds(16 * i, 16)] = zeros16
            return carry

        lax.fori_loop(0, _N_PAD // 16, zbody, 0)

        def sbody(k, carry):
            idx = didx_all[pl.ds(16 * k, 16)]
            plsc.addupdate_scatter(hist, [idx], ones16)
            return carry

        nfull = _EPT_DEG // 16
        lax.fori_loop(0, nfull, sbody, 0)
        rem = _EPT_DEG - nfull * 16
        if rem:
            idx = didx_all[pl.ds(16 * nfull, 16)]
            mask = lax.iota(jnp.int32, 16) < rem
            plsc.addupdate_scatter(hist, [jnp.where(mask, idx, 0)], ones16,
                                   mask=mask)
        pltpu.sync_copy(hist, stage_sh.at[s])
        plsc.subcore_barrier()

        @pl.when(s < _NRED)
        def _reduce():
            cols = _N_PAD // _NRED
            pltpu.sync_copy(stage_sh.at[:, pl.ds(s * cols, cols)], rbuf)

            def rbody(j, carry):
                acc = rbuf[0, pl.ds(16 * j, 16)]
                for r in range(1, _NS):
                    acc = acc + rbuf[r, pl.ds(16 * j, 16)]
                red[pl.ds(16 * j, 16)] = acc
                return carry

            lax.fori_loop(0, cols // 16, rbody, 0)
            pltpu.sync_copy(red, out_hbm.at[c, pl.ds(s * cols, cols)])

    @functools.partial(
        pl.kernel,
        out_type=jax.ShapeDtypeStruct((_NC, _N, _H), jnp.float32),
        mesh=mesh,
        scratch_types=[
            pltpu.VMEM((_EPT_MSG,), jnp.int32),
            pltpu.VMEM((_EPT_MSG,), jnp.int32),
            pltpu.VMEM((_CH_MSG,), jnp.int32),
            pltpu.VMEM((_CH_MSG,), jnp.int32),
            pltpu.VMEM((_CH_MSG, _H), jnp.float32),
            pltpu.VMEM((_CH_MSG, _H), jnp.float32),
            pltpu.VMEM_SHARED((_N, _H), jnp.float32),
            pltpu.SemaphoreType.DMA,
            pltpu.SemaphoreType.DMA,
            pltpu.SemaphoreType.DMA,
            pltpu.SemaphoreType.DMA,
        ],
    )
    def sc_msgpass(g_hbm, srcadj_hbm, dst_hbm, zeros_hbm, out_hbm,
                   sidx_all, didx_all, didx0, didx1, rows0, rows1, acc_sh,
                   sem0, sem1, sem0b, sem1b):
        c = lax.axis_index("c")
        s = lax.axis_index("s")

        @pl.when(s < _N_CH)
        def _zero():
            pltpu.sync_copy(zeros_hbm, acc_sh.at[pl.ds(s * _ROWS_CH, _ROWS_CH)])

        pltpu.sync_copy(srcadj_hbm.at[pl.ds(c * _E + s * _EPT_MSG, _EPT_MSG)],
                        sidx_all)
        pltpu.sync_copy(dst_hbm.at[pl.ds(s * _EPT_MSG, _EPT_MSG)], didx_all)
        plsc.subcore_barrier()

        def start_gather(chunk, rows, sem, semb):
            base = chunk * _CH_MSG
            sla = sidx_all.at[pl.ds(base, 48)]
            slb = sidx_all.at[pl.ds(base + 48, 32)]
            pltpu.async_copy(g_hbm.at[sla], rows.at[pl.ds(0, 48)], sem)
            pltpu.async_copy(g_hbm.at[slb], rows.at[pl.ds(48, 32)], semb)

        def wait_gather(chunk, rows, sem, semb):
            base = chunk * _CH_MSG
            sla = sidx_all.at[pl.ds(base, 48)]
            slb = sidx_all.at[pl.ds(base + 48, 32)]
            pltpu.make_async_copy(g_hbm.at[sla], rows.at[pl.ds(0, 48)],
                                  sem).wait()
            pltpu.make_async_copy(g_hbm.at[slb], rows.at[pl.ds(48, 32)],
                                  semb).wait()

        def stage_didx(chunk, didx):
            off = chunk * _CH_MSG
            for j in range(_CH_MSG // 16):
                didx[pl.ds(16 * j, 16)] = didx_all[pl.ds(off + 16 * j, 16)]

        stage_didx(0, didx0)
        start_gather(0, rows0, sem0, sem0b)

        def body(k2, carry):
            a = 2 * k2
            stage_didx(a + 1, didx1)
            start_gather(a + 1, rows1, sem1, sem1b)
            wait_gather(a, rows0, sem0, sem0b)
            pltpu.sync_copy(rows0, acc_sh.at[didx0], add=True)
            stage_didx(a + 2, didx0)
            start_gather(a + 2, rows0, sem0, sem0b)
            wait_gather(a + 1, rows1, sem1, sem1b)
            pltpu.sync_copy(rows1, acc_sh.at[didx1], add=True)
            return carry

        lax.fori_loop(0, (_NCHUNK - 1) // 2, body, 0)
        wait_gather(_NCHUNK - 1, rows0, sem0, sem0b)
        pltpu.sync_copy(rows0, acc_sh.at[didx0], add=True)
        plsc.subcore_barrier()

        @pl.when(s < _N_CH)
        def _writeback():
            pltpu.sync_copy(acc_sh.at[pl.ds(s * _ROWS_CH, _ROWS_CH)],
                            out_hbm.at[c, pl.ds(s * _ROWS_CH, _ROWS_CH)])

    return sc_degree, sc_msgpass


def _mm_body(x_ref, Wp_ref, bp_ref, h_ref):
    h = jnp.dot(x_ref[...], Wp_ref[...], preferred_element_type=jnp.float32)
    h_ref[...] = jnp.maximum(h + bp_ref[...], 0.0)


def _tc_premm(x, W_pre, b_pre):
    return pl.pallas_call(
        _mm_body,
        grid=(_N // _BN,),
        in_specs=[
            pl.BlockSpec((_BN, _D), lambda i: (i, 0)),
            pl.BlockSpec((_D, _D), lambda i: (0, 0)),
            pl.BlockSpec((1, _D), lambda i: (0, 0)),
        ],
        out_specs=pl.BlockSpec((_BN, _D), lambda i: (i, 0)),
        out_shape=jax.ShapeDtypeStruct((_N, _D), jnp.float32),
    )(x, W_pre, b_pre.reshape(1, _D))


def _g_body(deg_ref, h_ref, g_ref, dinv_ref):
    dd = deg_ref[...] + 1.0
    dv = lax.rsqrt(jnp.maximum(dd, 1.0))
    h = h_ref[...]
    g = h * dv
    g_ref[0, :, :] = g[:, :_H]
    g_ref[1, :, :] = g[:, _H:]
    dinv_ref[...] = dv


def _tc_g(deg_col, h):
    return pl.pallas_call(
        _g_body,
        grid=(_N // _BN,),
        in_specs=[
            pl.BlockSpec((_BN, 1), lambda i: (i, 0)),
            pl.BlockSpec((_BN, _D), lambda i: (i, 0)),
        ],
        out_specs=[
            pl.BlockSpec((_NC, _BN, _H), lambda i: (0, i, 0)),
            pl.BlockSpec((_BN, 1), lambda i: (i, 0)),
        ],
        out_shape=[
            jax.ShapeDtypeStruct((_NC, _N, _H), jnp.float32),
            jax.ShapeDtypeStruct((_N, 1), jnp.float32),
        ],
    )(deg_col, h)


def _layer_body(S_ref, h_ref, dinv_ref, W_ref, b_ref, h2_ref, g_ref):
    dv = dinv_ref[...]
    h = h_ref[...]
    dv2 = dv * dv
    z0 = S_ref[0] * dv + h[:, :_H] * dv2
    z1 = S_ref[1] * dv + h[:, _H:] * dv2
    acc = (jnp.dot(z0, W_ref[:_H, :], preferred_element_type=jnp.float32)
           + jnp.dot(z1, W_ref[_H:, :], preferred_element_type=jnp.float32))
    h2 = jnp.maximum(acc + b_ref[...], 0.0) + h
    h2_ref[...] = h2
    g = h2 * dv
    g_ref[0, :, :] = g[:, :_H]
    g_ref[1, :, :] = g[:, _H:]


def _tc_layer(S, h, dinv, W, b):
    return pl.pallas_call(
        _layer_body,
        grid=(_N // _BN,),
        in_specs=[
            pl.BlockSpec((_NC, _BN, _H), lambda i: (0, i, 0)),
            pl.BlockSpec((_BN, _D), lambda i: (i, 0)),
            pl.BlockSpec((_BN, 1), lambda i: (i, 0)),
            pl.BlockSpec((_D, _D), lambda i: (0, 0)),
            pl.BlockSpec((1, _D), lambda i: (0, 0)),
        ],
        out_specs=[
            pl.BlockSpec((_BN, _D), lambda i: (i, 0)),
            pl.BlockSpec((_NC, _BN, _H), lambda i: (0, i, 0)),
        ],
        out_shape=[
            jax.ShapeDtypeStruct((_N, _D), jnp.float32),
            jax.ShapeDtypeStruct((_NC, _N, _H), jnp.float32),
        ],
    )(S, h, dinv, W, b.reshape(1, _D))


def _tail_body(S_ref, h_ref, dinv_ref, W3_ref, b3_ref, Wh_ref, bh_ref, out_ref):
    dv = dinv_ref[...]
    h = h_ref[...]
    dv2 = dv * dv
    z0 = S_ref[0] * dv + h[:, :_H] * dv2
    z1 = S_ref[1] * dv + h[:, _H:] * dv2
    acc = (jnp.dot(z0, W3_ref[:_H, :], preferred_element_type=jnp.float32)
           + jnp.dot(z1, W3_ref[_H:, :], preferred_element_type=jnp.float32))
    h3 = jnp.maximum(acc + b3_ref[...], 0.0) + h
    out = jnp.dot(h3, Wh_ref[...], preferred_element_type=jnp.float32)
    out_ref[...] = out + bh_ref[...]


def _tc_tail(S, h, dinv, W3, b3, W_head, b_head):
    return pl.pallas_call(
        _tail_body,
        grid=(_N // _BN,),
        in_specs=[
            pl.BlockSpec((_NC, _BN, _H), lambda i: (0, i, 0)),
            pl.BlockSpec((_BN, _D), lambda i: (i, 0)),
            pl.BlockSpec((_BN, 1), lambda i: (i, 0)),
            pl.BlockSpec((_D, _D), lambda i: (0, 0)),
            pl.BlockSpec((1, _D), lambda i: (0, 0)),
            pl.BlockSpec((_D, _D), lambda i: (0, 0)),
            pl.BlockSpec((1, _D), lambda i: (0, 0)),
        ],
        out_specs=pl.BlockSpec((_BN, _D), lambda i: (i, 0)),
        out_shape=jax.ShapeDtypeStruct((_N, _D), jnp.float32),
    )(S, h, dinv, W3, b3.reshape(1, _D), W_head, b_head.reshape(1, _D))


def kernel(x, edge_index, W_pre, b_pre, W1, b1, W2, b2, W3, b3, W_head, b_head):
    src = edge_index[0]
    dst = edge_index[1]
    srcadj = jnp.concatenate([src, src + _N])
    zerosH = jnp.zeros((_ROWS_CH, _H), jnp.float32)

    sc_degree, sc_msgpass = _sc_kernels()
    deg_parts = sc_degree(dst)
    h = _tc_premm(x, W_pre, b_pre)
    deg_col = (deg_parts[0, :_N] + deg_parts[1, :_N]).reshape(_N, 1)
    g, dinv = _tc_g(deg_col, h)
    for W, b in ((W1, b1), (W2, b2)):
        S = sc_msgpass(g.reshape(_NC * _N, _H), srcadj, dst, zerosH)
        h, g = _tc_layer(S, h, dinv, W, b)
    S = sc_msgpass(g.reshape(_NC * _N, _H), srcadj, dst, zerosH)
    return _tc_tail(S, h, dinv, W3, b3, W_head, b_head)
